# Initial kernel scaffold; baseline (speedup 1.0000x reference)
import jax
import jax.numpy as jnp
from jax.experimental import pallas as pl


def kernel(x_batch, y_batch, eps, n_projections, theta_raw):
    raise NotImplementedError("write your pallas kernel here")


# speedup vs baseline: 3.0329x; 3.0329x over previous
import jax
import jax.numpy as jnp
from jax import lax
from jax.experimental import pallas as pl
from jax.experimental.pallas import tpu as pltpu
from jax.experimental.pallas import tpu_sc as plsc

B, N, D, P = 4, 16384, 64, 64
C = B * P
L = 16
V = N // L
RADIX = 256
CNT = RADIX * L
NC, NS = 2, 16
NW = NC * NS
CPW = C // NW
NT = 2048

_SIGN = -2**31


def _normalize_theta(th):
    norm = jnp.sqrt(jnp.sum(th * th, axis=1, keepdims=True))
    return th / jnp.maximum(norm, 1e-8)


def _proj_body(x_ref, y_ref, th_ref, xo_ref, yo_ref):
    th = _normalize_theta(th_ref[...])
    dn = (((1,), (1,)), ((), ()))
    xo_ref[0] = lax.dot_general(th, x_ref[0], dn,
                                preferred_element_type=jnp.float32,
                                precision=lax.Precision.HIGHEST)
    yo_ref[0] = lax.dot_general(th, y_ref[0], dn,
                                preferred_element_type=jnp.float32,
                                precision=lax.Precision.HIGHEST)


def _project(x, y, theta_raw):
    grid = (B, N // NT)
    return pl.pallas_call(
        _proj_body,
        grid=grid,
        in_specs=[
            pl.BlockSpec((1, NT, D), lambda b, n: (b, n, 0)),
            pl.BlockSpec((1, NT, D), lambda b, n: (b, n, 0)),
            pl.BlockSpec((P, D), lambda b, n: (0, 0)),
        ],
        out_specs=[
            pl.BlockSpec((1, P, NT), lambda b, n: (b, 0, n)),
            pl.BlockSpec((1, P, NT), lambda b, n: (b, 0, n)),
        ],
        out_shape=[
            jax.ShapeDtypeStruct((B, P, N), jnp.float32),
            jax.ShapeDtypeStruct((B, P, N), jnp.float32),
        ],
    )(x, y, theta_raw)


def _encode(u):
    m = lax.shift_right_arithmetic(u, 31)
    return u ^ (m | _SIGN)


def _decode(e):
    m = lax.shift_right_arithmetic(e, 31)
    return e ^ (~m | _SIGN)


def _sc_body(x_hbm, y_hbm, out_hbm, xb, yb, ak, av, bk, bv, cnt):
    wid = lax.axis_index("s") * NC + lax.axis_index("c")
    lane = lax.iota(jnp.int32, L)

    def radix_pass(load_key, srcv, dstk, dstv, shift):
        def zero(i, _):
            cnt[pl.ds(i * L, L)] = jnp.zeros((L,), jnp.int32)
            return 0
        lax.fori_loop(0, CNT // L, zero, 0)

        def hist(v, _):
            k = load_key(v)
            d = lax.shift_right_logical(k, shift) & 255
            idx = d * L + lane
            c = plsc.load_gather(cnt, [idx])
            plsc.store_scatter(cnt, [idx], c + 1)
            return 0
        lax.fori_loop(0, V, hist, 0)

        def scan(i, carry):
            c = cnt[pl.ds(i * L, L)]
            s = plsc.cumsum(c)
            cnt[pl.ds(i * L, L)] = s - c + carry
            return carry + jnp.sum(c)
        lax.fori_loop(0, CNT // L, scan, jnp.int32(0))

        def perm(v, _):
            k = load_key(v)
            d = lax.shift_right_logical(k, shift) & 255
            idx = d * L + lane
            o = plsc.load_gather(cnt, [idx])
            plsc.store_scatter(cnt, [idx], o + 1)
            mem = (o & (V - 1)) * L + lax.shift_right_logical(o, 10)
            plsc.store_scatter(dstk, [mem], k)
            if dstv is not None:
                val = (v * L + lane) if srcv is None else srcv[pl.ds(v * L, L)]
                plsc.store_scatter(dstv, [mem], val)
            return 0
        lax.fori_loop(0, V, perm, 0)

    def enc_from(ref):
        return lambda v: _encode(ref[pl.ds(v * L, L)])

    def raw_from(ref):
        return lambda v: ref[pl.ds(v * L, L)]

    def column(j, _):
        col = wid * CPW + j
        pltpu.sync_copy(x_hbm.at[col], xb)
        pltpu.sync_copy(y_hbm.at[col], yb)
        radix_pass(enc_from(xb), None, ak, av, 0)
        radix_pass(raw_from(ak), av, bk, bv, 8)
        radix_pass(raw_from(bk), bv, ak, av, 16)
        radix_pass(raw_from(ak), av, bk, bv, 24)
        radix_pass(enc_from(yb), None, ak, None, 0)
        radix_pass(raw_from(ak), None, av, None, 8)
        radix_pass(raw_from(av), None, ak, None, 16)
        radix_pass(raw_from(ak), None, av, None, 24)
        def comb(v, _):
            sl = pl.ds(v * L, L)
            plsc.store_scatter(xb, [bv[sl]], _decode(av[sl]))
            return 0
        lax.fori_loop(0, V, comb, 0)
        pltpu.sync_copy(xb, out_hbm.at[col])
        return 0

    lax.fori_loop(0, CPW, column, 0)


def _sc_transport(xT_bits, yT_bits):
    mesh = plsc.VectorSubcoreMesh(core_axis_name="c", subcore_axis_name="s",
                                  num_cores=NC, num_subcores=NS)
    f = pl.kernel(
        _sc_body,
        out_type=jax.ShapeDtypeStruct((C, N), jnp.int32),
        mesh=mesh,
        compiler_params=pltpu.CompilerParams(needs_layout_passes=False),
        scratch_types=[
            pltpu.VMEM((N,), jnp.int32),
            pltpu.VMEM((N,), jnp.int32),
            pltpu.VMEM((N,), jnp.int32),
            pltpu.VMEM((N,), jnp.int32),
            pltpu.VMEM((N,), jnp.int32),
            pltpu.VMEM((N,), jnp.int32),
            pltpu.VMEM((CNT,), jnp.int32),
        ],
    )
    return f(xT_bits, yT_bits)


def _assemble_body(scale_ref, t_ref, xp_ref, x_ref, th_ref, o_ref):
    th = _normalize_theta(th_ref[...])
    transported = lax.bitcast_convert_type(t_ref[0], jnp.float32)
    diff = transported - xp_ref[0]
    dn = (((0,), (0,)), ((), ()))
    t = lax.dot_general(diff, th, dn,
                        preferred_element_type=jnp.float32,
                        precision=lax.Precision.HIGHEST)
    o_ref[0] = x_ref[0] + t * scale_ref[0]


def _assemble(transT_bits, xT, x, theta_raw, n_projections):
    grid = (B, N // NT)
    scale = (1.0 / jnp.asarray(n_projections, jnp.float32)).reshape(1)
    return pl.pallas_call(
        _assemble_body,
        grid=grid,
        in_specs=[
            pl.BlockSpec(memory_space=pltpu.SMEM),
            pl.BlockSpec((1, P, NT), lambda b, n: (b, 0, n)),
            pl.BlockSpec((1, P, NT), lambda b, n: (b, 0, n)),
            pl.BlockSpec((1, NT, D), lambda b, n: (b, n, 0)),
            pl.BlockSpec((P, D), lambda b, n: (0, 0)),
        ],
        out_specs=pl.BlockSpec((1, NT, D), lambda b, n: (b, n, 0)),
        out_shape=jax.ShapeDtypeStruct((B, N, D), jnp.float32),
    )(scale, transT_bits, xT, x, theta_raw)


def kernel(x_batch, y_batch, eps, n_projections, theta_raw):
    del eps
    xT, yT = _project(x_batch, y_batch, theta_raw)
    xT_bits = lax.bitcast_convert_type(xT, jnp.int32).reshape(C, N)
    yT_bits = lax.bitcast_convert_type(yT, jnp.int32).reshape(C, N)
    transT_bits = _sc_transport(xT_bits, yT_bits).reshape(B, P, N)
    return _assemble(transT_bits, xT, x_batch, theta_raw, n_projections)


# speedup vs baseline: 3.7647x; 1.2413x over previous
import jax
import jax.numpy as jnp
from jax import lax
from jax.experimental import pallas as pl
from jax.experimental.pallas import tpu as pltpu
from jax.experimental.pallas import tpu_sc as plsc

B, N, D, P = 4, 16384, 64, 64
C = B * P
L = 16
V = N // L
RADIX = 256
CNT = RADIX * L
NC, NS = 2, 16
NW = NC * NS
CPW = C // NW
NT = 2048

_SIGN = -2**31


def _normalize_theta(th):
    norm = jnp.sqrt(jnp.sum(th * th, axis=1, keepdims=True))
    return th / jnp.maximum(norm, 1e-8)


def _proj_body(x_ref, y_ref, th_ref, xo_ref, yo_ref):
    th = _normalize_theta(th_ref[...])
    dn = (((1,), (1,)), ((), ()))
    xo_ref[0] = lax.dot_general(th, x_ref[0], dn,
                                preferred_element_type=jnp.float32,
                                precision=lax.Precision.HIGHEST)
    yo_ref[0] = lax.dot_general(th, y_ref[0], dn,
                                preferred_element_type=jnp.float32,
                                precision=lax.Precision.HIGHEST)


def _project(x, y, theta_raw):
    grid = (B, N // NT)
    return pl.pallas_call(
        _proj_body,
        grid=grid,
        in_specs=[
            pl.BlockSpec((1, NT, D), lambda b, n: (b, n, 0)),
            pl.BlockSpec((1, NT, D), lambda b, n: (b, n, 0)),
            pl.BlockSpec((P, D), lambda b, n: (0, 0)),
        ],
        out_specs=[
            pl.BlockSpec((1, P, NT), lambda b, n: (b, 0, n)),
            pl.BlockSpec((1, P, NT), lambda b, n: (b, 0, n)),
        ],
        out_shape=[
            jax.ShapeDtypeStruct((B, P, N), jnp.float32),
            jax.ShapeDtypeStruct((B, P, N), jnp.float32),
        ],
    )(x, y, theta_raw)


def _encode(u):
    m = lax.shift_right_arithmetic(u, 31)
    return u ^ (m | _SIGN)


def _decode(e):
    m = lax.shift_right_arithmetic(e, 31)
    return e ^ (~m | _SIGN)


def _sc_body(x_hbm, y_hbm, out_hbm, xb, yb, ak, av, bk, bv, ck, cntx, cnty):
    wid = lax.axis_index("s") * NC + lax.axis_index("c")
    lane = lax.iota(jnp.int32, L)
    ones = jnp.ones((L,), jnp.int32)
    zeros = jnp.zeros((L,), jnp.int32)

    def pass_xy(xk_src, xv_src, xk_dst, xv_dst, yk_src, yk_dst, shift, first):
        def loadx(v):
            k = xk_src[pl.ds(v * L, L)]
            return _encode(k) if first else k

        def loady(v):
            k = yk_src[pl.ds(v * L, L)]
            return _encode(k) if first else k

        def zero(i, _):
            for u in range(2):
                sl = pl.ds((i * 2 + u) * L, L)
                cntx[sl] = zeros
                cnty[sl] = zeros
            return 0
        lax.fori_loop(0, CNT // L // 2, zero, 0)

        def hist(i, _):
            for u in range(2):
                v = i * 2 + u
                kx = loadx(v)
                dx = lax.shift_right_logical(kx, shift) & 255
                plsc.addupdate_scatter(cntx, [dx * L + lane], ones)
                ky = loady(v)
                dy = lax.shift_right_logical(ky, shift) & 255
                plsc.addupdate_scatter(cnty, [dy * L + lane], ones)
            return 0
        lax.fori_loop(0, V // 2, hist, 0)

        def scan(i, carry):
            cax, cay = carry
            for u in range(2):
                sl = pl.ds((i * 2 + u) * L, L)
                cx = cntx[sl]
                sx = plsc.cumsum(cx)
                cntx[sl] = sx - cx + cax
                cax = cax + jnp.sum(cx)
                cy = cnty[sl]
                sy = plsc.cumsum(cy)
                cnty[sl] = sy - cy + cay
                cay = cay + jnp.sum(cy)
            return cax, cay
        lax.fori_loop(0, CNT // L // 2, scan, (jnp.int32(0), jnp.int32(0)))

        def perm(i, _):
            for u in range(2):
                v = i * 2 + u
                kx = loadx(v)
                dx = lax.shift_right_logical(kx, shift) & 255
                ix = dx * L + lane
                ox = plsc.load_gather(cntx, [ix])
                plsc.store_scatter(cntx, [ix], ox + 1)
                memx = (ox & (V - 1)) * L + lax.shift_right_logical(ox, 10)
                plsc.store_scatter(xk_dst, [memx], kx)
                val = (v * L + lane) if first else xv_src[pl.ds(v * L, L)]
                plsc.store_scatter(xv_dst, [memx], val)
                ky = loady(v)
                dy = lax.shift_right_logical(ky, shift) & 255
                iy = dy * L + lane
                oy = plsc.load_gather(cnty, [iy])
                plsc.store_scatter(cnty, [iy], oy + 1)
                memy = (oy & (V - 1)) * L + lax.shift_right_logical(oy, 10)
                plsc.store_scatter(yk_dst, [memy], ky)
            return 0
        lax.fori_loop(0, V // 2, perm, 0)

    def column(j, _):
        col = wid * CPW + j
        pltpu.sync_copy(x_hbm.at[col], xb)
        pltpu.sync_copy(y_hbm.at[col], yb)
        pass_xy(xb, None, ak, av, yb, ck, 0, True)
        pass_xy(ak, av, bk, bv, ck, yb, 8, False)
        pass_xy(bk, bv, ak, av, yb, ck, 16, False)
        pass_xy(ak, av, bk, bv, ck, yb, 24, False)
        def comb(i, _):
            for u in range(4):
                sl = pl.ds((i * 4 + u) * L, L)
                plsc.store_scatter(xb, [bv[sl]], _decode(yb[sl]))
            return 0
        lax.fori_loop(0, V // 4, comb, 0)
        pltpu.sync_copy(xb, out_hbm.at[col])
        return 0

    lax.fori_loop(0, CPW, column, 0)


def _sc_transport(xT_bits, yT_bits):
    mesh = plsc.VectorSubcoreMesh(core_axis_name="c", subcore_axis_name="s",
                                  num_cores=NC, num_subcores=NS)
    f = pl.kernel(
        _sc_body,
        out_type=jax.ShapeDtypeStruct((C, N), jnp.int32),
        mesh=mesh,
        compiler_params=pltpu.CompilerParams(needs_layout_passes=False),
        scratch_types=[
            pltpu.VMEM((N,), jnp.int32),
            pltpu.VMEM((N,), jnp.int32),
            pltpu.VMEM((N,), jnp.int32),
            pltpu.VMEM((N,), jnp.int32),
            pltpu.VMEM((N,), jnp.int32),
            pltpu.VMEM((N,), jnp.int32),
            pltpu.VMEM((N,), jnp.int32),
            pltpu.VMEM((CNT,), jnp.int32),
            pltpu.VMEM((CNT,), jnp.int32),
        ],
    )
    return f(xT_bits, yT_bits)


def _assemble_body(scale_ref, t_ref, xp_ref, x_ref, th_ref, o_ref):
    th = _normalize_theta(th_ref[...])
    transported = lax.bitcast_convert_type(t_ref[0], jnp.float32)
    diff = transported - xp_ref[0]
    dn = (((0,), (0,)), ((), ()))
    t = lax.dot_general(diff, th, dn,
                        preferred_element_type=jnp.float32,
                        precision=lax.Precision.HIGHEST)
    o_ref[0] = x_ref[0] + t * scale_ref[0]


def _assemble(transT_bits, xT, x, theta_raw, n_projections):
    grid = (B, N // NT)
    scale = (1.0 / jnp.asarray(n_projections, jnp.float32)).reshape(1)
    return pl.pallas_call(
        _assemble_body,
        grid=grid,
        in_specs=[
            pl.BlockSpec(memory_space=pltpu.SMEM),
            pl.BlockSpec((1, P, NT), lambda b, n: (b, 0, n)),
            pl.BlockSpec((1, P, NT), lambda b, n: (b, 0, n)),
            pl.BlockSpec((1, NT, D), lambda b, n: (b, n, 0)),
            pl.BlockSpec((P, D), lambda b, n: (0, 0)),
        ],
        out_specs=pl.BlockSpec((1, NT, D), lambda b, n: (b, n, 0)),
        out_shape=jax.ShapeDtypeStruct((B, N, D), jnp.float32),
    )(scale, transT_bits, xT, x, theta_raw)


def kernel(x_batch, y_batch, eps, n_projections, theta_raw):
    del eps
    xT, yT = _project(x_batch, y_batch, theta_raw)
    xT_bits = lax.bitcast_convert_type(xT, jnp.int32).reshape(C, N)
    yT_bits = lax.bitcast_convert_type(yT, jnp.int32).reshape(C, N)
    transT_bits = _sc_transport(xT_bits, yT_bits).reshape(B, P, N)
    return _assemble(transT_bits, xT, x_batch, theta_raw, n_projections)


# speedup vs baseline: 5.0391x; 1.3385x over previous
import jax
import jax.numpy as jnp
from jax import lax
from jax.experimental import pallas as pl
from jax.experimental.pallas import tpu as pltpu
from jax.experimental.pallas import tpu_sc as plsc

B, N, D, P = 4, 16384, 64, 64
C = B * P
L = 16
V = N // L
RADIX = 256
CNT = RADIX * L
PART = 2
H = V // PART
NC, NS = 2, 16
NW = NC * NS
CPW = C // NW
NT = 2048

_SIGN = -2**31


def _normalize_theta(th):
    norm = jnp.sqrt(jnp.sum(th * th, axis=1, keepdims=True))
    return th / jnp.maximum(norm, 1e-8)


def _proj_body(x_ref, y_ref, th_ref, xo_ref, yo_ref):
    th = _normalize_theta(th_ref[...])
    dn = (((1,), (1,)), ((), ()))
    xo_ref[0] = lax.dot_general(th, x_ref[0], dn,
                                preferred_element_type=jnp.float32,
                                precision=lax.Precision.HIGHEST)
    yo_ref[0] = lax.dot_general(th, y_ref[0], dn,
                                preferred_element_type=jnp.float32,
                                precision=lax.Precision.HIGHEST)


def _project(x, y, theta_raw):
    grid = (B, N // NT)
    return pl.pallas_call(
        _proj_body,
        grid=grid,
        in_specs=[
            pl.BlockSpec((1, NT, D), lambda b, n: (b, n, 0)),
            pl.BlockSpec((1, NT, D), lambda b, n: (b, n, 0)),
            pl.BlockSpec((P, D), lambda b, n: (0, 0)),
        ],
        out_specs=[
            pl.BlockSpec((1, P, NT), lambda b, n: (b, 0, n)),
            pl.BlockSpec((1, P, NT), lambda b, n: (b, 0, n)),
        ],
        out_shape=[
            jax.ShapeDtypeStruct((B, P, N), jnp.float32),
            jax.ShapeDtypeStruct((B, P, N), jnp.float32),
        ],
    )(x, y, theta_raw)


def _encode(u):
    m = lax.shift_right_arithmetic(u, 31)
    return u ^ (m | _SIGN)


def _decode(e):
    m = lax.shift_right_arithmetic(e, 31)
    return e ^ (~m | _SIGN)


def _sc_body(x_hbm, y_hbm, out_hbm, ak, av, bk, bv, yb, ck, cntx, cnty):
    wid = lax.axis_index("s") * NC + lax.axis_index("c")
    lane = lax.iota(jnp.int32, L)
    ones = jnp.ones((L,), jnp.int32)
    zeros = jnp.zeros((L,), jnp.int32)

    def rank_to_mem(r):
        return (r & (V - 1)) * L + lax.shift_right_logical(r, 10)

    def radix_pass(xk_src, xv_src, xv_dst, yk_src, shift, first, last,
                   xk_dst=None, yk_dst=None, out=None):
        def loadx(v):
            k = xk_src[pl.ds(v * L, L)]
            return _encode(k) if first else k

        def loady(v):
            k = yk_src[pl.ds(v * L, L)]
            return _encode(k) if first else k

        @plsc.parallel_loop(0, PART * CNT // L, unroll=4)
        def _zero(i):
            sl = pl.ds(i * L, L)
            cntx[sl] = zeros
            cnty[sl] = zeros

        @plsc.parallel_loop(0, H, unroll=2)
        def _hist(i):
            for p in range(PART):
                v = i + p * H
                kx = loadx(v)
                dx = lax.shift_right_logical(kx, shift) & 255
                plsc.addupdate_scatter(cntx, [p * CNT + dx * L + lane], ones)
                ky = loady(v)
                dy = lax.shift_right_logical(ky, shift) & 255
                plsc.addupdate_scatter(cnty, [p * CNT + dy * L + lane], ones)

        @plsc.parallel_loop(0, CNT // L, carry=(jnp.int32(0), jnp.int32(0)))
        def _scan(i, carry):
            cax, cay = carry
            sl0 = pl.ds(i * L, L)
            sl1 = pl.ds(CNT + i * L, L)
            cx0 = cntx[sl0]
            cx1 = cntx[sl1]
            sx = cx0 + cx1
            px = plsc.cumsum(sx) - sx + cax
            cntx[sl0] = px
            cntx[sl1] = px + cx0
            cax = cax + jnp.sum(sx)
            cy0 = cnty[sl0]
            cy1 = cnty[sl1]
            sy = cy0 + cy1
            py = plsc.cumsum(sy) - sy + cay
            cnty[sl0] = py
            cnty[sl1] = py + cy0
            cay = cay + jnp.sum(sy)
            return cax, cay

        def permx(v, p, store_keys):
            kx = loadx(v)
            dx = lax.shift_right_logical(kx, shift) & 255
            ix = p * CNT + dx * L + lane
            ox = plsc.load_gather(cntx, [ix])
            plsc.store_scatter(cntx, [ix], ox + 1)
            memx = rank_to_mem(ox)
            if store_keys:
                plsc.store_scatter(xk_dst, [memx], kx)
            val = (v * L + lane) if first else xv_src[pl.ds(v * L, L)]
            plsc.store_scatter(xv_dst, [memx], val)

        def permy(v, p):
            ky = loady(v)
            dy = lax.shift_right_logical(ky, shift) & 255
            iy = p * CNT + dy * L + lane
            oy = plsc.load_gather(cnty, [iy])
            plsc.store_scatter(cnty, [iy], oy + 1)
            memy = rank_to_mem(oy)
            if not last:
                plsc.store_scatter(yk_dst, [memy], ky)
            else:
                pos = plsc.load_gather(xv_dst, [memy])
                plsc.store_scatter(out, [pos], _decode(ky))

        if not last:
            def perm(i, _):
                for p in range(PART):
                    v = i + p * H
                    permx(v, p, True)
                    permy(v, p)
                return 0
            lax.fori_loop(0, H, perm, 0)
        else:
            def perm_x(i, _):
                for p in range(PART):
                    permx(i + p * H, p, False)
                return 0
            lax.fori_loop(0, H, perm_x, 0)

            def perm_y(i, _):
                for p in range(PART):
                    permy(i + p * H, p)
                return 0
            lax.fori_loop(0, H, perm_y, 0)

    def column(j, _):
        col = wid * CPW + j
        pltpu.sync_copy(x_hbm.at[col], ak)
        pltpu.sync_copy(y_hbm.at[col], yb)
        radix_pass(ak, None, bv, yb, 0, True, False, xk_dst=bk, yk_dst=ck)
        radix_pass(bk, bv, av, ck, 8, False, False, xk_dst=ak, yk_dst=yb)
        radix_pass(ak, av, bv, yb, 16, False, False, xk_dst=bk, yk_dst=ck)
        radix_pass(bk, bv, av, ck, 24, False, True, out=ak)
        pltpu.sync_copy(ak, out_hbm.at[col])
        return 0

    lax.fori_loop(0, CPW, column, 0)


def _sc_transport(xT_bits, yT_bits):
    mesh = plsc.VectorSubcoreMesh(core_axis_name="c", subcore_axis_name="s",
                                  num_cores=NC, num_subcores=NS)
    f = pl.kernel(
        _sc_body,
        out_type=jax.ShapeDtypeStruct((C, N), jnp.int32),
        mesh=mesh,
        compiler_params=pltpu.CompilerParams(needs_layout_passes=False),
        scratch_types=[
            pltpu.VMEM((N,), jnp.int32),
            pltpu.VMEM((N,), jnp.int32),
            pltpu.VMEM((N,), jnp.int32),
            pltpu.VMEM((N,), jnp.int32),
            pltpu.VMEM((N,), jnp.int32),
            pltpu.VMEM((N,), jnp.int32),
            pltpu.VMEM((PART * CNT,), jnp.int32),
            pltpu.VMEM((PART * CNT,), jnp.int32),
        ],
    )
    return f(xT_bits, yT_bits)


def _assemble_body(scale_ref, t_ref, xp_ref, x_ref, th_ref, o_ref):
    th = _normalize_theta(th_ref[...])
    transported = lax.bitcast_convert_type(t_ref[0], jnp.float32)
    diff = transported - xp_ref[0]
    dn = (((0,), (0,)), ((), ()))
    t = lax.dot_general(diff, th, dn,
                        preferred_element_type=jnp.float32,
                        precision=lax.Precision.HIGHEST)
    o_ref[0] = x_ref[0] + t * scale_ref[0]


def _assemble(transT_bits, xT, x, theta_raw, n_projections):
    grid = (B, N // NT)
    scale = (1.0 / jnp.asarray(n_projections, jnp.float32)).reshape(1)
    return pl.pallas_call(
        _assemble_body,
        grid=grid,
        in_specs=[
            pl.BlockSpec(memory_space=pltpu.SMEM),
            pl.BlockSpec((1, P, NT), lambda b, n: (b, 0, n)),
            pl.BlockSpec((1, P, NT), lambda b, n: (b, 0, n)),
            pl.BlockSpec((1, NT, D), lambda b, n: (b, n, 0)),
            pl.BlockSpec((P, D), lambda b, n: (0, 0)),
        ],
        out_specs=pl.BlockSpec((1, NT, D), lambda b, n: (b, n, 0)),
        out_shape=jax.ShapeDtypeStruct((B, N, D), jnp.float32),
    )(scale, transT_bits, xT, x, theta_raw)


def kernel(x_batch, y_batch, eps, n_projections, theta_raw):
    del eps
    xT, yT = _project(x_batch, y_batch, theta_raw)
    xT_bits = lax.bitcast_convert_type(xT, jnp.int32).reshape(C, N)
    yT_bits = lax.bitcast_convert_type(yT, jnp.int32).reshape(C, N)
    transT_bits = _sc_transport(xT_bits, yT_bits).reshape(B, P, N)
    return _assemble(transT_bits, xT, x_batch, theta_raw, n_projections)
